# baseline (speedup 1.0000x reference)
import functools

import jax
import jax.numpy as jnp
from jax import lax
from jax.experimental import pallas as pl
from jax.experimental.pallas import tpu as pltpu
from jax.experimental.pallas import tpu_sc as plsc

NC, NS, L = 2, 16, 16
NW = NC * NS
G = 128
PAD = 8
ZR = 40


@functools.lru_cache(maxsize=None)
def _make_segsum(T, N, E, D):
    EPW = E // NW
    assert EPW * NW == E
    NCH = EPW // G
    assert NCH * G == EPW and NCH % 2 == 0
    NA = N + PAD
    NCHK = N // ZR
    assert NCHK * ZR == N
    IT = (NCHK + NS - 1) // NS
    mesh = plsc.VectorSubcoreMesh(core_axis_name="c", subcore_axis_name="s")

    @functools.partial(
        pl.kernel,
        out_type=jax.ShapeDtypeStruct((T, NC, N, D), jnp.float32),
        mesh=mesh,
        scratch_types=[
            pltpu.VMEM((NCH, G), jnp.int32),
            pltpu.VMEM((NCH, G), jnp.int32),
            pltpu.VMEM((G, D), jnp.float32),
            pltpu.VMEM((G, D), jnp.float32),
            pltpu.VMEM((ZR, D), jnp.float32),
            pltpu.VMEM_SHARED((NA, D), jnp.float32),
            pltpu.SemaphoreType.DMA,
            pltpu.SemaphoreType.DMA,
        ],
    )
    def k(x_hbm, src_hbm, dst_hbm, out_hbm, sidx, didx, rows_a, rows_b,
          zbuf, acc, sem_a, sem_b):
        c = lax.axis_index("c")
        s = lax.axis_index("s")
        wid = s * NC + c
        zero = jnp.zeros((L,), jnp.float32)

        pltpu.sync_copy(src_hbm.at[wid], sidx)
        pltpu.sync_copy(dst_hbm.at[wid], didx)

        def zb(r, _):
            for j in range(D // L):
                zbuf[r, pl.ds(j * L, L)] = zero
            return 0

        lax.fori_loop(0, ZR, zb, 0)

        def tbody(t, _):
            @pl.when(t > 0)
            def _():
                def bump(i, _):
                    r = i // (G // L)
                    co = (i % (G // L)) * L
                    sidx[r, pl.ds(co, L)] = sidx[r, pl.ds(co, L)] + N
                    return 0

                lax.fori_loop(0, NCH * (G // L), bump, 0)

            def za(i, _):
                ch = s + i * NS

                @pl.when(ch < NCHK)
                def _():
                    pltpu.sync_copy(zbuf, acc.at[pl.ds(ch * ZR, ZR)])

                return 0

            lax.fori_loop(0, IT, za, 0)
            plsc.subcore_barrier()

            def body(i, _):
                pltpu.async_copy(x_hbm.at[sidx.at[i]], rows_a, sem_a).wait()
                pltpu.sync_copy(rows_a, acc.at[didx.at[i]], add=True)
                return 0

            lax.fori_loop(0, NCH, body, 0)
            plsc.subcore_barrier()

            def wo(i, _):
                ch = s + i * NS

                @pl.when(ch < NCHK)
                def _():
                    pltpu.sync_copy(acc.at[pl.ds(ch * ZR, ZR)],
                                    out_hbm.at[t, c, pl.ds(ch * ZR, ZR)])

                return 0

            lax.fori_loop(0, IT, wo, 0)
            plsc.subcore_barrier()
            return 0

        lax.fori_loop(0, T, tbody, 0)

    return k


@functools.lru_cache(maxsize=None)
def _make_tc_layer(T, N, D):
    R = 512

    def body(x_ref, pa_ref, pb_ref, da_ref, db_ref, ws_ref, wn_ref, b_ref, o_ref):
        deg = da_ref[0, :, 0:1] + db_ref[0, :, 0:1]
        inv = 1.0 / jnp.maximum(deg, 1.0)
        nm = (pa_ref[0, 0] + pb_ref[0, 0]) * inv
        acc = jnp.dot(x_ref[0], ws_ref[...], preferred_element_type=jnp.float32)
        acc = acc + jnp.dot(nm, wn_ref[...], preferred_element_type=jnp.float32)
        o_ref[0] = jnp.maximum(acc + b_ref[...], 0.0)

    return pl.pallas_call(
        body,
        grid=(T, pl.cdiv(N, R)),
        in_specs=[
            pl.BlockSpec((1, R, D), lambda t, i: (t, i, 0)),
            pl.BlockSpec((1, 1, R, D), lambda t, i: (t, 0, i, 0)),
            pl.BlockSpec((1, 1, R, D), lambda t, i: (t, 1, i, 0)),
            pl.BlockSpec((1, R, D), lambda t, i: (0, i, 0)),
            pl.BlockSpec((1, R, D), lambda t, i: (1, i, 0)),
            pl.BlockSpec((D, D), lambda t, i: (0, 0)),
            pl.BlockSpec((D, D), lambda t, i: (0, 0)),
            pl.BlockSpec((1, D), lambda t, i: (0, 0)),
        ],
        out_specs=pl.BlockSpec((1, R, D), lambda t, i: (t, i, 0)),
        out_shape=jax.ShapeDtypeStruct((T, N, D), jnp.float32),
    )


def kernel(inputs, edge_index, W1s, W1n, b1, W2s, W2n, b2, W3s, W3n, b3):
    T, N, D = inputs.shape
    E = edge_index.shape[1]
    NCH = -(-E // (NW * G))
    NCH += NCH % 2
    EP = NW * NCH * G
    src = jnp.concatenate([edge_index[0], jnp.zeros((EP - E,), jnp.int32)])
    dst = jnp.concatenate([edge_index[1], jnp.full((EP - E,), N, jnp.int32)])
    src = src.reshape(NW, NCH, G)
    dst = dst.reshape(NW, NCH, G)

    segsum = _make_segsum(T, N, EP, D)
    segsum1 = _make_segsum(1, N, EP, D)
    tc = _make_tc_layer(T, N, D)

    degP = segsum1(jnp.ones((N, D), jnp.float32), src, dst)[0]
    layers = [
        (W1s, W1n, b1.reshape(1, D)),
        (W2s, W2n, b2.reshape(1, D)),
        (W3s, W3n, b3.reshape(1, D)),
    ]
    h = inputs
    for Ws_, Wn_, b_ in layers:
        P = segsum(h.reshape(T * N, D), src, dst)
        h = tc(h, P, P, degP, degP, Ws_, Wn_, b_)
    return h


# speedup vs baseline: 1.0109x; 1.0109x over previous
import functools

import jax
import jax.numpy as jnp
from jax import lax
from jax.experimental import pallas as pl
from jax.experimental.pallas import tpu as pltpu
from jax.experimental.pallas import tpu_sc as plsc

NC, NS, L = 2, 16, 16
NW = NC * NS
G = 128
PAD = 8
ZR = 40


@functools.lru_cache(maxsize=None)
def _make_segsum(T, N, E, D):
    EPW = E // NW
    assert EPW * NW == E
    NCH = EPW // G
    assert NCH * G == EPW and NCH % 2 == 0
    NA = N + PAD
    NCHK = N // ZR
    assert NCHK * ZR == N
    IT = (NCHK + NS - 1) // NS
    mesh = plsc.VectorSubcoreMesh(core_axis_name="c", subcore_axis_name="s")

    @functools.partial(
        pl.kernel,
        out_type=jax.ShapeDtypeStruct((T, NC, N, D), jnp.float32),
        mesh=mesh,
        scratch_types=[
            pltpu.VMEM((NCH, G), jnp.int32),
            pltpu.VMEM((NCH, G), jnp.int32),
            pltpu.VMEM((G, D), jnp.float32),
            pltpu.VMEM((G, D), jnp.float32),
            pltpu.VMEM((ZR, D), jnp.float32),
            pltpu.VMEM_SHARED((NA, D), jnp.float32),
            pltpu.SemaphoreType.DMA,
            pltpu.SemaphoreType.DMA,
        ],
    )
    def k(x_hbm, src_hbm, dst_hbm, out_hbm, sidx, didx, rows_a, rows_b,
          zbuf, acc, sem_a, sem_b):
        c = lax.axis_index("c")
        s = lax.axis_index("s")
        wid = s * NC + c
        zero = jnp.zeros((L,), jnp.float32)

        pltpu.sync_copy(src_hbm.at[wid], sidx)
        pltpu.sync_copy(dst_hbm.at[wid], didx)

        def zb(r, _):
            for j in range(D // L):
                zbuf[r, pl.ds(j * L, L)] = zero
            return 0

        lax.fori_loop(0, ZR, zb, 0)

        def tbody(t, _):
            @pl.when(t > 0)
            def _():
                def bump(i, _):
                    r = i // (G // L)
                    co = (i % (G // L)) * L
                    sidx[r, pl.ds(co, L)] = sidx[r, pl.ds(co, L)] + N
                    return 0

                lax.fori_loop(0, NCH * (G // L), bump, 0)

            def za(i, _):
                ch = s + i * NS

                @pl.when(ch < NCHK)
                def _():
                    pltpu.sync_copy(zbuf, acc.at[pl.ds(ch * ZR, ZR)])

                return 0

            lax.fori_loop(0, IT, za, 0)
            plsc.subcore_barrier()

            def body(i, _):
                pltpu.async_copy(x_hbm.at[sidx.at[i]], rows_a, sem_a).wait()
                pltpu.sync_copy(rows_a, acc.at[didx.at[i]], add=True)
                return 0

            lax.fori_loop(0, NCH, body, 0)
            plsc.subcore_barrier()

            def wo(i, _):
                ch = s + i * NS

                @pl.when(ch < NCHK)
                def _():
                    pltpu.sync_copy(acc.at[pl.ds(ch * ZR, ZR)], zbuf)
                    pltpu.sync_copy(zbuf, out_hbm.at[t, c, pl.ds(ch * ZR, ZR)])

                return 0

            lax.fori_loop(0, IT, wo, 0)
            plsc.subcore_barrier()
            return 0

        lax.fori_loop(0, T, tbody, 0)

    return k


@functools.lru_cache(maxsize=None)
def _make_tc_layer(T, N, D):
    R = 512

    def body(x_ref, pa_ref, pb_ref, da_ref, db_ref, ws_ref, wn_ref, b_ref, o_ref):
        deg = da_ref[0, :, 0:1] + db_ref[0, :, 0:1]
        inv = 1.0 / jnp.maximum(deg, 1.0)
        nm = (pa_ref[0, 0] + pb_ref[0, 0]) * inv
        acc = jnp.dot(x_ref[0], ws_ref[...], preferred_element_type=jnp.float32)
        acc = acc + jnp.dot(nm, wn_ref[...], preferred_element_type=jnp.float32)
        o_ref[0] = jnp.maximum(acc + b_ref[...], 0.0)

    return pl.pallas_call(
        body,
        grid=(T, pl.cdiv(N, R)),
        in_specs=[
            pl.BlockSpec((1, R, D), lambda t, i: (t, i, 0)),
            pl.BlockSpec((1, 1, R, D), lambda t, i: (t, 0, i, 0)),
            pl.BlockSpec((1, 1, R, D), lambda t, i: (t, 1, i, 0)),
            pl.BlockSpec((1, R, D), lambda t, i: (0, i, 0)),
            pl.BlockSpec((1, R, D), lambda t, i: (1, i, 0)),
            pl.BlockSpec((D, D), lambda t, i: (0, 0)),
            pl.BlockSpec((D, D), lambda t, i: (0, 0)),
            pl.BlockSpec((1, D), lambda t, i: (0, 0)),
        ],
        out_specs=pl.BlockSpec((1, R, D), lambda t, i: (t, i, 0)),
        out_shape=jax.ShapeDtypeStruct((T, N, D), jnp.float32),
    )


def kernel(inputs, edge_index, W1s, W1n, b1, W2s, W2n, b2, W3s, W3n, b3):
    T, N, D = inputs.shape
    E = edge_index.shape[1]
    NCH = -(-E // (NW * G))
    NCH += NCH % 2
    EP = NW * NCH * G
    src = jnp.concatenate([edge_index[0], jnp.zeros((EP - E,), jnp.int32)])
    dst = jnp.concatenate([edge_index[1], jnp.full((EP - E,), N, jnp.int32)])
    src = src.reshape(NW, NCH, G)
    dst = dst.reshape(NW, NCH, G)

    segsum = _make_segsum(T, N, EP, D)
    segsum1 = _make_segsum(1, N, EP, D)
    tc = _make_tc_layer(T, N, D)

    degP = segsum1(jnp.ones((N, D), jnp.float32), src, dst)[0]
    layers = [
        (W1s, W1n, b1.reshape(1, D)),
        (W2s, W2n, b2.reshape(1, D)),
        (W3s, W3n, b3.reshape(1, D)),
    ]
    h = inputs
    for Ws_, Wn_, b_ in layers:
        P = segsum(h.reshape(T * N, D), src, dst)
        h = tc(h, P, P, degP, degP, Ws_, Wn_, b_)
    return h


# speedup vs baseline: 1.9133x; 1.8926x over previous
import functools

import jax
import jax.numpy as jnp
from jax import lax
from jax.experimental import pallas as pl
from jax.experimental.pallas import tpu as pltpu
from jax.experimental.pallas import tpu_sc as plsc

NC, NS, L = 2, 16, 16
NW = NC * NS
G = 80


@functools.lru_cache(maxsize=None)
def _make_gather_segsum(N, E, D):
    EPW = E // NW
    assert EPW * NW == E and EPW % G == 0
    NCH = EPW // G
    ZR = 200
    NCHK = N // ZR
    assert NCHK * ZR == N
    IT = (NCHK + NS - 1) // NS
    mesh = plsc.VectorSubcoreMesh(core_axis_name="c", subcore_axis_name="s")

    @functools.partial(
        pl.kernel,
        out_type=jax.ShapeDtypeStruct((NC, N, D), jnp.float32),
        mesh=mesh,
        scratch_types=[
            pltpu.VMEM((G,), jnp.int32),
            pltpu.VMEM((G,), jnp.int32),
            pltpu.VMEM((G, D), jnp.float32),
            pltpu.VMEM((ZR, D), jnp.float32),
            pltpu.VMEM_SHARED((N, D), jnp.float32),
            pltpu.SemaphoreType.DMA,
        ],
    )
    def k(x_hbm, src_hbm, dst_hbm, out_hbm, sidx, didx, rows, zbuf, acc, sem):
        c = lax.axis_index("c")
        s = lax.axis_index("s")
        wid = s * NC + c
        zero = jnp.zeros((L,), jnp.float32)

        def zb(r, _):
            for j in range(D // L):
                zbuf[r, pl.ds(j * L, L)] = zero
            return 0

        lax.fori_loop(0, ZR, zb, 0)

        def za(i, _):
            ch = s + i * NS

            @pl.when(ch < NCHK)
            def _():
                pltpu.sync_copy(zbuf, acc.at[pl.ds(ch * ZR, ZR)])

            return 0

        lax.fori_loop(0, IT, za, 0)
        plsc.subcore_barrier()

        base = wid * EPW

        def body(i, _):
            off = base + i * G
            pltpu.sync_copy(src_hbm.at[pl.ds(off, G)], sidx)
            pltpu.sync_copy(dst_hbm.at[pl.ds(off, G)], didx)
            pltpu.async_copy(x_hbm.at[sidx], rows, sem).wait()
            pltpu.sync_copy(rows, acc.at[didx], add=True)
            return 0

        lax.fori_loop(0, NCH, body, 0)
        plsc.subcore_barrier()

        def wo(i, _):
            ch = s + i * NS

            @pl.when(ch < NCHK)
            def _():
                pltpu.sync_copy(acc.at[pl.ds(ch * ZR, ZR)], zbuf)
                pltpu.sync_copy(zbuf, out_hbm.at[c, pl.ds(ch * ZR, ZR)])

            return 0

        lax.fori_loop(0, IT, wo, 0)

    return k


@functools.lru_cache(maxsize=None)
def _make_deg(N, E):
    D = 128
    EPW = E // NW
    NCH = EPW // G
    ZR = 200
    NCHK = N // ZR
    assert NCHK * ZR == N
    IT = (NCHK + NS - 1) // NS
    mesh = plsc.VectorSubcoreMesh(core_axis_name="c", subcore_axis_name="s")

    @functools.partial(
        pl.kernel,
        out_type=jax.ShapeDtypeStruct((NC, N, D), jnp.float32),
        mesh=mesh,
        scratch_types=[
            pltpu.VMEM((G,), jnp.int32),
            pltpu.VMEM((G, D), jnp.float32),
            pltpu.VMEM((ZR, D), jnp.float32),
            pltpu.VMEM_SHARED((N, D), jnp.float32),
        ],
    )
    def k(dst_hbm, out_hbm, didx, ones, zbuf, acc):
        c = lax.axis_index("c")
        s = lax.axis_index("s")
        wid = s * NC + c
        zero = jnp.zeros((L,), jnp.float32)
        one = jnp.ones((L,), jnp.float32)

        def zb(r, _):
            for j in range(D // L):
                zbuf[r, pl.ds(j * L, L)] = zero
            return 0

        lax.fori_loop(0, ZR, zb, 0)

        def ob(r, _):
            for j in range(D // L):
                ones[r, pl.ds(j * L, L)] = one
            return 0

        lax.fori_loop(0, G, ob, 0)

        def za(i, _):
            ch = s + i * NS

            @pl.when(ch < NCHK)
            def _():
                pltpu.sync_copy(zbuf, acc.at[pl.ds(ch * ZR, ZR)])

            return 0

        lax.fori_loop(0, IT, za, 0)
        plsc.subcore_barrier()

        base = wid * EPW

        def body(i, _):
            pltpu.sync_copy(dst_hbm.at[pl.ds(base + i * G, G)], didx)
            pltpu.sync_copy(ones, acc.at[didx], add=True)
            return 0

        lax.fori_loop(0, NCH, body, 0)
        plsc.subcore_barrier()

        def wo(i, _):
            ch = s + i * NS

            @pl.when(ch < NCHK)
            def _():
                pltpu.sync_copy(acc.at[pl.ds(ch * ZR, ZR)], zbuf)
                pltpu.sync_copy(zbuf, out_hbm.at[c, pl.ds(ch * ZR, ZR)])

            return 0

        lax.fori_loop(0, IT, wo, 0)

    return k


@functools.lru_cache(maxsize=None)
def _make_tc_layer(N, D):
    R = 512

    def body(x_ref, pa_ref, pb_ref, da_ref, db_ref, ws_ref, wn_ref, b_ref, o_ref):
        deg = da_ref[0, :, 0:1] + db_ref[0, :, 0:1]
        inv = 1.0 / jnp.maximum(deg, 1.0)
        nm = (pa_ref[0] + pb_ref[0]) * inv
        acc = jnp.dot(x_ref[...], ws_ref[...], preferred_element_type=jnp.float32)
        acc = acc + jnp.dot(nm, wn_ref[...], preferred_element_type=jnp.float32)
        o_ref[...] = jnp.maximum(acc + b_ref[...], 0.0)

    return pl.pallas_call(
        body,
        grid=(pl.cdiv(N, R),),
        in_specs=[
            pl.BlockSpec((R, D), lambda i: (i, 0)),
            pl.BlockSpec((1, R, D), lambda i: (0, i, 0)),
            pl.BlockSpec((1, R, D), lambda i: (1, i, 0)),
            pl.BlockSpec((1, R, D), lambda i: (0, i, 0)),
            pl.BlockSpec((1, R, D), lambda i: (1, i, 0)),
            pl.BlockSpec((D, D), lambda i: (0, 0)),
            pl.BlockSpec((D, D), lambda i: (0, 0)),
            pl.BlockSpec((1, D), lambda i: (0, 0)),
        ],
        out_specs=pl.BlockSpec((R, D), lambda i: (i, 0)),
        out_shape=jax.ShapeDtypeStruct((N, D), jnp.float32),
    )


def kernel(inputs, edge_index, W1s, W1n, b1, W2s, W2n, b2, W3s, W3n, b3):
    T, N, D = inputs.shape
    E = edge_index.shape[1]
    src = edge_index[0]
    dst = edge_index[1]

    segsum = _make_gather_segsum(N, E, D)
    degk = _make_deg(N, E)
    tc = _make_tc_layer(N, D)

    degP = degk(dst)
    layers = [
        (W1s, W1n, b1.reshape(1, D)),
        (W2s, W2n, b2.reshape(1, D)),
        (W3s, W3n, b3.reshape(1, D)),
    ]
    outs = []
    for t in range(T):
        h = inputs[t]
        for Ws_, Wn_, b_ in layers:
            P = segsum(h, src, dst)
            h = tc(h, P, P, degP, degP, Ws_, Wn_, b_)
        outs.append(h)
    return jnp.stack(outs, axis=0)


# speedup vs baseline: 2.7989x; 1.4629x over previous
import functools

import jax
import jax.numpy as jnp
from jax import lax
from jax.experimental import pallas as pl
from jax.experimental.pallas import tpu as pltpu
from jax.experimental.pallas import tpu_sc as plsc

NC, NS, L = 2, 16, 16
NW = NC * NS
G = 80


@functools.lru_cache(maxsize=None)
def _make_gather_segsum(N, E, D):
    EPW = E // NW
    assert EPW * NW == E and EPW % G == 0
    NCH = EPW // G
    ZR = 200
    NCHK = N // ZR
    assert NCHK * ZR == N
    IT = (NCHK + NS - 1) // NS
    mesh = plsc.VectorSubcoreMesh(core_axis_name="c", subcore_axis_name="s")

    @functools.partial(
        pl.kernel,
        out_type=jax.ShapeDtypeStruct((NC, N, D), jnp.float32),
        mesh=mesh,
        scratch_types=[
            pltpu.VMEM((G,), jnp.int32),
            pltpu.VMEM((G,), jnp.int32),
            pltpu.VMEM((G,), jnp.int32),
            pltpu.VMEM((G,), jnp.int32),
            pltpu.VMEM((G, D), jnp.float32),
            pltpu.VMEM((ZR, D), jnp.float32),
            pltpu.VMEM_SHARED((N, D), jnp.float32),
            pltpu.SemaphoreType.DMA,
            pltpu.SemaphoreType.DMA,
            pltpu.SemaphoreType.DMA,
        ],
    )
    def k(x_hbm, src_hbm, dst_hbm, out_hbm, sidx_a, didx_a, sidx_b, didx_b,
          rows, zbuf, acc, sem, sem_ia, sem_ib):
        c = lax.axis_index("c")
        s = lax.axis_index("s")
        wid = s * NC + c
        zero = jnp.zeros((L,), jnp.float32)

        def zb(r, _):
            for j in range(D // L):
                zbuf[r, pl.ds(j * L, L)] = zero
            return 0

        lax.fori_loop(0, ZR, zb, 0)

        def za(i, _):
            ch = s + i * NS

            @pl.when(ch < NCHK)
            def _():
                pltpu.sync_copy(zbuf, acc.at[pl.ds(ch * ZR, ZR)])

            return 0

        lax.fori_loop(0, IT, za, 0)
        plsc.subcore_barrier()

        base = wid * EPW

        pltpu.async_copy(src_hbm.at[pl.ds(base, G)], sidx_a, sem_ia)
        pltpu.async_copy(dst_hbm.at[pl.ds(base, G)], didx_a, sem_ia)

        def body(i, _):
            j0 = 2 * i
            off_b = base + (j0 + 1) * G
            pltpu.make_async_copy(src_hbm.at[pl.ds(base, G)], sidx_a, sem_ia).wait()
            pltpu.make_async_copy(dst_hbm.at[pl.ds(base, G)], didx_a, sem_ia).wait()
            pltpu.async_copy(src_hbm.at[pl.ds(off_b, G)], sidx_b, sem_ib)
            pltpu.async_copy(dst_hbm.at[pl.ds(off_b, G)], didx_b, sem_ib)
            pltpu.async_copy(x_hbm.at[sidx_a], rows, sem).wait()
            pltpu.sync_copy(rows, acc.at[didx_a], add=True)
            pltpu.make_async_copy(src_hbm.at[pl.ds(base, G)], sidx_b, sem_ib).wait()
            pltpu.make_async_copy(dst_hbm.at[pl.ds(base, G)], didx_b, sem_ib).wait()

            @pl.when(j0 + 2 < NCH)
            def _():
                off_a = base + (j0 + 2) * G
                pltpu.async_copy(src_hbm.at[pl.ds(off_a, G)], sidx_a, sem_ia)
                pltpu.async_copy(dst_hbm.at[pl.ds(off_a, G)], didx_a, sem_ia)

            pltpu.async_copy(x_hbm.at[sidx_b], rows, sem).wait()
            pltpu.sync_copy(rows, acc.at[didx_b], add=True)
            return 0

        lax.fori_loop(0, NCH // 2, body, 0)
        plsc.subcore_barrier()

        def wo(i, _):
            ch = s + i * NS

            @pl.when(ch < NCHK)
            def _():
                pltpu.sync_copy(acc.at[pl.ds(ch * ZR, ZR)], zbuf)
                pltpu.sync_copy(zbuf, out_hbm.at[c, pl.ds(ch * ZR, ZR)])

            return 0

        lax.fori_loop(0, IT, wo, 0)

    return k


@functools.lru_cache(maxsize=None)
def _make_deg(N, E):
    D = 128
    EPW = E // NW
    NCH = EPW // G
    ZR = 200
    NCHK = N // ZR
    assert NCHK * ZR == N
    IT = (NCHK + NS - 1) // NS
    mesh = plsc.VectorSubcoreMesh(core_axis_name="c", subcore_axis_name="s")

    @functools.partial(
        pl.kernel,
        out_type=jax.ShapeDtypeStruct((NC, N, D), jnp.float32),
        mesh=mesh,
        scratch_types=[
            pltpu.VMEM((G,), jnp.int32),
            pltpu.VMEM((G, D), jnp.float32),
            pltpu.VMEM((ZR, D), jnp.float32),
            pltpu.VMEM_SHARED((N, D), jnp.float32),
        ],
    )
    def k(dst_hbm, out_hbm, didx, ones, zbuf, acc):
        c = lax.axis_index("c")
        s = lax.axis_index("s")
        wid = s * NC + c
        zero = jnp.zeros((L,), jnp.float32)
        one = jnp.ones((L,), jnp.float32)

        def zb(r, _):
            for j in range(D // L):
                zbuf[r, pl.ds(j * L, L)] = zero
            return 0

        lax.fori_loop(0, ZR, zb, 0)

        def ob(r, _):
            for j in range(D // L):
                ones[r, pl.ds(j * L, L)] = one
            return 0

        lax.fori_loop(0, G, ob, 0)

        def za(i, _):
            ch = s + i * NS

            @pl.when(ch < NCHK)
            def _():
                pltpu.sync_copy(zbuf, acc.at[pl.ds(ch * ZR, ZR)])

            return 0

        lax.fori_loop(0, IT, za, 0)
        plsc.subcore_barrier()

        base = wid * EPW

        def body(i, _):
            pltpu.sync_copy(dst_hbm.at[pl.ds(base + i * G, G)], didx)
            pltpu.sync_copy(ones, acc.at[didx], add=True)
            return 0

        lax.fori_loop(0, NCH, body, 0)
        plsc.subcore_barrier()

        def wo(i, _):
            ch = s + i * NS

            @pl.when(ch < NCHK)
            def _():
                pltpu.sync_copy(acc.at[pl.ds(ch * ZR, ZR)], zbuf)
                pltpu.sync_copy(zbuf, out_hbm.at[c, pl.ds(ch * ZR, ZR)])

            return 0

        lax.fori_loop(0, IT, wo, 0)

    return k


@functools.lru_cache(maxsize=None)
def _make_tc_layer(N, D):
    R = 512

    def body(x_ref, pa_ref, pb_ref, da_ref, db_ref, ws_ref, wn_ref, b_ref, o_ref):
        deg = da_ref[0, :, 0:1] + db_ref[0, :, 0:1]
        inv = 1.0 / jnp.maximum(deg, 1.0)
        nm = (pa_ref[0] + pb_ref[0]) * inv
        acc = jnp.dot(x_ref[...], ws_ref[...], preferred_element_type=jnp.float32)
        acc = acc + jnp.dot(nm, wn_ref[...], preferred_element_type=jnp.float32)
        o_ref[...] = jnp.maximum(acc + b_ref[...], 0.0)

    return pl.pallas_call(
        body,
        grid=(pl.cdiv(N, R),),
        in_specs=[
            pl.BlockSpec((R, D), lambda i: (i, 0)),
            pl.BlockSpec((1, R, D), lambda i: (0, i, 0)),
            pl.BlockSpec((1, R, D), lambda i: (1, i, 0)),
            pl.BlockSpec((1, R, D), lambda i: (0, i, 0)),
            pl.BlockSpec((1, R, D), lambda i: (1, i, 0)),
            pl.BlockSpec((D, D), lambda i: (0, 0)),
            pl.BlockSpec((D, D), lambda i: (0, 0)),
            pl.BlockSpec((1, D), lambda i: (0, 0)),
        ],
        out_specs=pl.BlockSpec((R, D), lambda i: (i, 0)),
        out_shape=jax.ShapeDtypeStruct((N, D), jnp.float32),
    )


def kernel(inputs, edge_index, W1s, W1n, b1, W2s, W2n, b2, W3s, W3n, b3):
    T, N, D = inputs.shape
    E = edge_index.shape[1]
    src = edge_index[0]
    dst = edge_index[1]

    segsum = _make_gather_segsum(N, E, D)
    degk = _make_deg(N, E)
    tc = _make_tc_layer(N, D)

    degP = degk(dst)
    layers = [
        (W1s, W1n, b1.reshape(1, D)),
        (W2s, W2n, b2.reshape(1, D)),
        (W3s, W3n, b3.reshape(1, D)),
    ]
    outs = []
    for t in range(T):
        h = inputs[t]
        for Ws_, Wn_, b_ in layers:
            P = segsum(h, src, dst)
            h = tc(h, P, P, degP, degP, Ws_, Wn_, b_)
        outs.append(h)
    return jnp.stack(outs, axis=0)
